# baseline (speedup 1.0000x reference)
import jax
import jax.numpy as jnp
from jax import lax
from jax.experimental import pallas as pl
from jax.experimental.pallas import tpu as pltpu

H = 3
HID = 16
N = 400
F = 144
E = 3200
TB = 96
BBLK = 8
NSTEP = TB // BBLK


def _dense_h_kernel(in_ref, cov_ref, wd_ref, wm_ref, ws_ref,
                    ald_ref, alm_ref, als_ref, ard_ref, arm_ref, ars_ref,
                    hd_ref, hm_ref, hs_ref,
                    eld_ref, elm_ref, els_ref, erd_ref, erm_ref, ers_ref):
    x = in_ref[...] + cov_ref[...]
    for w_ref, al_ref, ar_ref, h_ref, el_ref, er_ref in (
            (wd_ref, ald_ref, ard_ref, hd_ref, eld_ref, erd_ref),
            (wm_ref, alm_ref, arm_ref, hm_ref, elm_ref, erm_ref),
            (ws_ref, als_ref, ars_ref, hs_ref, els_ref, ers_ref)):
        h = jnp.dot(x, w_ref[...], preferred_element_type=jnp.float32)
        h_ref[...] = h
        el_ref[...] = jnp.dot(h, al_ref[...], preferred_element_type=jnp.float32)
        er_ref[...] = jnp.dot(h, ar_ref[...], preferred_element_type=jnp.float32)


def _gat_kernel(src_ref, dst_ref, el_ref, er_ref, h_ref, out_ref, alpha_ref):
    step = pl.program_id(0)
    n_iota = lax.broadcasted_iota(jnp.int32, (E, N), 1)
    gs = (src_ref[...] == n_iota).astype(jnp.bfloat16)
    gd = (dst_ref[...] == n_iota).astype(jnp.bfloat16)

    @pl.when(step == 0)
    def _():
        els = jnp.dot(gs, el_ref[...].astype(jnp.bfloat16),
                      preferred_element_type=jnp.float32)
        erd = jnp.dot(gd, er_ref[...].astype(jnp.bfloat16),
                      preferred_element_type=jnp.float32)
        e = els + erd
        e = jnp.where(e >= 0, e, 0.2 * e)
        m = jnp.max(e, axis=0, keepdims=True)
        ee = jnp.exp(e - m)
        esum = lax.dot_general(gd, ee.astype(jnp.bfloat16),
                               (((0,), (0,)), ((), ())),
                               preferred_element_type=jnp.float32)
        esum_e = jnp.dot(gd, esum.astype(jnp.bfloat16),
                         preferred_element_type=jnp.float32)
        alpha = ee / (esum_e + 1e-9)
        for jj in range(NSTEP):
            alpha_ref[jj] = alpha[:, jj * (BBLK * H):(jj + 1) * (BBLK * H)]

    @pl.when(step > 0)
    def _():
        j = step - 1
        hcat = jnp.concatenate(
            [h_ref[b].astype(jnp.bfloat16) for b in range(BBLK)], axis=1)
        z = jnp.dot(gs, hcat, preferred_element_type=jnp.float32)
        a_iota = lax.broadcasted_iota(jnp.int32, (BBLK * H, BBLK * 48), 0)
        c_iota = lax.broadcasted_iota(jnp.int32, (BBLK * H, BBLK * 48), 1)
        ex = (c_iota // HID == a_iota).astype(jnp.float32)
        ae = jnp.dot(alpha_ref[j], ex, preferred_element_type=jnp.float32)
        zz = (z * ae).astype(jnp.bfloat16)
        out_ref[...] = lax.dot_general(gd, zz, (((0,), (0,)), ((), ())),
                                       preferred_element_type=jnp.float32)


def _attn_kernel(cov_ref, out_ref, acc_ref):
    t = pl.program_id(1)

    @pl.when(t == 0)
    def _():
        c = cov_ref[0]
        a = lax.dot_general(c, c, (((1,), (1,)), ((), ())),
                            preferred_element_type=jnp.float32)
        m = jnp.max(a, axis=1, keepdims=True)
        p = jnp.exp(a - m)
        acc_ref[...] = p / jnp.sum(p, axis=1, keepdims=True)

    out_ref[0] = acc_ref[...]


def _expand_al(al):
    flat = al.reshape(48)
    cols = jnp.arange(48) // HID
    onehot = (jnp.arange(8)[None, :] == cols[:, None]).astype(jnp.float32)
    return flat[:, None] * onehot


def _gat_edge(src2, dst2, elT, erT, h3):
    return pl.pallas_call(
        _gat_kernel,
        grid=(NSTEP + 1,),
        out_shape=jax.ShapeDtypeStruct((N, TB * 48), jnp.float32),
        in_specs=[
            pl.BlockSpec((E, 1), lambda s: (0, 0)),
            pl.BlockSpec((E, 1), lambda s: (0, 0)),
            pl.BlockSpec((N, TB * H), lambda s: (0, 0)),
            pl.BlockSpec((N, TB * H), lambda s: (0, 0)),
            pl.BlockSpec((BBLK, N, 48), lambda s: (jnp.maximum(s - 1, 0), 0, 0)),
        ],
        out_specs=pl.BlockSpec((N, BBLK * 48), lambda s: (0, jnp.maximum(s - 1, 0))),
        scratch_shapes=[pltpu.VMEM((NSTEP, E, BBLK * H), jnp.float32)],
    )(src2, dst2, elT, erT, h3)


def kernel(input, covariate, edge_index_d, W_d, al_d, ar_d, b_d,
           edge_index_m, W_m, al_m, ar_m, b_m,
           edge_index_s, W_s, al_s, ar_s, b_s):
    batch, T = input.shape[0], input.shape[1]
    in_r = input.reshape(TB * N, F)
    cov_r = covariate.reshape(TB * N, F)

    blk = N * BBLK
    grid_a = (TB * N) // blk
    als = [_expand_al(a) for a in (al_d, al_m, al_s)]
    ars = [_expand_al(a) for a in (ar_d, ar_m, ar_s)]
    row_spec = pl.BlockSpec((blk, F), lambda i: (i, 0))
    w_spec = pl.BlockSpec((F, 48), lambda i: (0, 0))
    a_spec = pl.BlockSpec((48, 8), lambda i: (0, 0))
    h_spec = pl.BlockSpec((blk, 48), lambda i: (i, 0))
    e_spec = pl.BlockSpec((blk, 8), lambda i: (i, 0))
    res = pl.pallas_call(
        _dense_h_kernel,
        grid=(grid_a,),
        out_shape=[jax.ShapeDtypeStruct((TB * N, 48), jnp.float32)] * 3
        + [jax.ShapeDtypeStruct((TB * N, 8), jnp.float32)] * 6,
        in_specs=[row_spec, row_spec, w_spec, w_spec, w_spec,
                  a_spec, a_spec, a_spec, a_spec, a_spec, a_spec],
        out_specs=[h_spec] * 3 + [e_spec] * 6,
    )(in_r, cov_r, W_d, W_m, W_s, *als, *ars)
    h_gs = res[:3]
    el_gs = res[3:6]
    er_gs = res[6:9]

    outs = []
    for g, (edge_index, b) in enumerate([(edge_index_d, b_d),
                                         (edge_index_m, b_m),
                                         (edge_index_s, b_s)]):
        src2 = edge_index[0].astype(jnp.int32).reshape(E, 1)
        dst2 = edge_index[1].astype(jnp.int32).reshape(E, 1)
        h3 = h_gs[g].reshape(TB, N, 48)
        elT = el_gs[g].reshape(TB, N, 8)[:, :, :H].transpose(1, 0, 2).reshape(N, TB * H)
        erT = er_gs[g].reshape(TB, N, 8)[:, :, :H].transpose(1, 0, 2).reshape(N, TB * H)
        out_g = _gat_edge(src2, dst2, elT, erT, h3)
        outs.append(out_g.reshape(N, TB, H, HID) + b.reshape(1, 1, H, HID))

    x_attn = jnp.concatenate(outs, axis=-1)
    x_attn = jnp.transpose(x_attn, (1, 0, 2, 3)).reshape(batch, T, N, F)
    x_attn = jnp.transpose(x_attn, (1, 0, 2, 3)).reshape(batch, T, N, F)
    out = input + x_attn

    cov0 = covariate[:, 0]
    attn = pl.pallas_call(
        _attn_kernel,
        grid=(batch, T),
        out_shape=jax.ShapeDtypeStruct((batch * T, N, N), jnp.float32),
        in_specs=[pl.BlockSpec((1, N, F), lambda bb, t: (bb, 0, 0))],
        out_specs=pl.BlockSpec((1, N, N), lambda bb, t: (t * batch + bb, 0, 0)),
        scratch_shapes=[pltpu.VMEM((N, N), jnp.float32)],
    )(cov0)
    return out, attn
